# baseline (speedup 1.0000x reference)
import jax
import jax.numpy as jnp
from jax import lax
from jax.experimental import pallas as pl

_N = 5000
_NPAD = 5120
_T = 128
_NT = _NPAD // _T
_TOPN = 2000
_TH = 0.7
_IM = 512.0
_WIN = 256
_PW = 1024
_OUT_ROWS = 2304

_DN = (((1,), (0,)), ((), ()))
_HI = lax.Precision.HIGHEST


def _iou_mask(x1c, y1c, x2c, y2c, ac, x1r, y1r, x2r, y2r, ar):
    xx1 = jnp.maximum(x1c, x1r)
    yy1 = jnp.maximum(y1c, y1r)
    xx2 = jnp.minimum(x2c, x2r)
    yy2 = jnp.minimum(y2c, y2r)
    w = jnp.clip(xx2 - xx1 + 1.0, 0.0)
    h = jnp.clip(yy2 - yy1 + 1.0, 0.0)
    inter = w * h
    iou = inter / (ac + ar - inter)
    return iou > _TH


def _iou_gt(x1c, y1c, x2c, y2c, ac, x1r, y1r, x2r, y2r, ar):
    return _iou_mask(x1c, y1c, x2c, y2c, ac, x1r, y1r, x2r, y2r,
                     ar).astype(jnp.float32)


def _nms_kernel(x1_ref, y1_ref, x2_ref, y2_ref, s_ref, out_ref):
    out_ref[...] = jnp.zeros((_OUT_ROWS, 8), jnp.float32)

    ii = lax.broadcasted_iota(jnp.int32, (_T, _T), 0)
    jj = lax.broadcasted_iota(jnp.int32, (_T, _T), 1)
    strict_ut = (ii < jj).astype(jnp.float32)
    incl_ut = (ii <= jj).astype(jnp.float32)
    ident = (ii == jj).astype(jnp.float32)
    row_iota = lax.broadcasted_iota(jnp.int32, (_WIN, 1), 0).astype(
        jnp.float32)
    lane_iota = lax.broadcasted_iota(jnp.int32, (1, _T), 1)

    def tile_body(t, base):
        off = t * _T
        xt1 = jnp.clip(x1_ref[:, pl.ds(off, _T)], 0.0, _IM - 1.0)
        yt1 = jnp.clip(y1_ref[:, pl.ds(off, _T)], 0.0, _IM - 1.0)
        xt2 = jnp.clip(x2_ref[:, pl.ds(off, _T)], 0.0, _IM - 1.0)
        yt2 = jnp.clip(y2_ref[:, pl.ds(off, _T)], 0.0, _IM - 1.0)
        at = (xt2 - xt1 + 1.0) * (yt2 - yt1 + 1.0)
        st = s_ref[:, pl.ds(off, _T)]

        stacked8 = jnp.concatenate(
            [st, xt1, yt1, xt2, yt2, at, jnp.zeros((2, _T), jnp.float32)],
            axis=0)
        cols8 = lax.dot_general(ident, stacked8, (((1,), (1,)), ((), ())),
                                preferred_element_type=jnp.float32,
                                precision=_HI)
        x1c = cols8[:, 1:2]
        y1c = cols8[:, 2:3]
        x2c = cols8[:, 3:4]
        y2c = cols8[:, 4:5]
        ac = cols8[:, 5:6]

        def pre_body(c, acc):
            poff = c * _PW
            px1 = out_ref[pl.ds(poff, _PW), 1:2]
            py1 = out_ref[pl.ds(poff, _PW), 2:3]
            px2 = out_ref[pl.ds(poff, _PW), 3:4]
            py2 = out_ref[pl.ds(poff, _PW), 4:5]
            pa = out_ref[pl.ds(poff, _PW), 5:6]
            s_c = _iou_mask(px1, py1, px2, py2, pa, xt1, yt1, xt2, yt2, at)
            return jnp.where(jnp.any(s_c, axis=0, keepdims=True), 1.0, acc)

        sup = lax.fori_loop(0, (base + _PW - 1) // _PW, pre_body,
                            jnp.zeros((1, _T), jnp.float32))
        a_mask = jnp.where(
            (sup > 0.0) | ((lane_iota + off) >= _N), 0.0, 1.0)

        s_mat = _iou_gt(x1c, y1c, x2c, y2c, ac, xt1, yt1, xt2, yt2, at)
        s_mat = s_mat * strict_ut

        def fp_cond(c):
            return c[1]

        def fp_once(k):
            sup_k = lax.dot_general(k, s_mat, _DN,
                                    preferred_element_type=jnp.float32,
                                    precision=_HI)
            return jnp.where(sup_k > 0.0, 0.0, a_mask)

        def fp_body(c):
            k, _ = c
            nk = fp_once(fp_once(k))
            return nk, jnp.any(nk != k)

        keep, _ = lax.while_loop(fp_cond, fp_body, (a_mask, jnp.bool_(True)))

        pos_incl = lax.dot_general(keep, incl_ut, _DN,
                                   preferred_element_type=jnp.float32,
                                   precision=_HI)
        cnt = jnp.sum(keep).astype(jnp.int32)
        posf = base.astype(jnp.float32) + pos_incl - 1.0

        base_al = (jnp.minimum(base, _TOPN) // 8) * 8
        rel = posf - base_al.astype(jnp.float32)
        oh = ((row_iota == rel) & (keep > 0.0)
              & (posf < float(_TOPN))).astype(jnp.float32)
        upd = lax.dot_general(oh, cols8, _DN,
                              preferred_element_type=jnp.float32,
                              precision=_HI)
        cur = out_ref[pl.ds(base_al, _WIN), :]
        out_ref[pl.ds(base_al, _WIN), :] = cur + upd
        return base + cnt

    def tile_step(t, base):
        return lax.cond(base < _TOPN, lambda: tile_body(t, base),
                        lambda: base)

    lax.fori_loop(0, _NT, tile_step, jnp.int32(0))


def kernel(boxes, scores):
    _, x1, y1, x2, y2, s = jax.lax.sort(
        (-scores, boxes[:, 0], boxes[:, 1], boxes[:, 2], boxes[:, 3],
         scores),
        num_keys=1, is_stable=True)
    pad = _NPAD - _N
    x1 = jnp.pad(x1, ((0, pad),))[None, :]
    y1 = jnp.pad(y1, ((0, pad),))[None, :]
    x2 = jnp.pad(x2, ((0, pad),))[None, :]
    y2 = jnp.pad(y2, ((0, pad),))[None, :]
    s = jnp.pad(s, ((0, pad),))[None, :]
    out = pl.pallas_call(
        _nms_kernel,
        out_shape=jax.ShapeDtypeStruct((_OUT_ROWS, 8), jnp.float32),
    )(x1, y1, x2, y2, s)
    return out[:_TOPN, :5]


# speedup vs baseline: 1.0070x; 1.0070x over previous
import jax
import jax.numpy as jnp
from jax import lax
from jax.experimental import pallas as pl

_N = 5000
_NPAD = 5120
_T = 128
_NT = _NPAD // _T
_TOPN = 2000
_TH = 0.7
_IM = 512.0
_WIN = 136
_PW = 256
_OUT_ROWS = 2304

_DN = (((1,), (0,)), ((), ()))
_HI = lax.Precision.HIGHEST


def _iou_mask(x1c, y1c, x2c, y2c, ac, x1r, y1r, x2r, y2r, ar):
    xx1 = jnp.maximum(x1c, x1r)
    yy1 = jnp.maximum(y1c, y1r)
    xx2 = jnp.minimum(x2c, x2r)
    yy2 = jnp.minimum(y2c, y2r)
    w = jnp.clip(xx2 - xx1 + 1.0, 0.0)
    h = jnp.clip(yy2 - yy1 + 1.0, 0.0)
    inter = w * h
    iou = inter / (ac + ar - inter)
    return iou > _TH


def _iou_gt(x1c, y1c, x2c, y2c, ac, x1r, y1r, x2r, y2r, ar):
    return _iou_mask(x1c, y1c, x2c, y2c, ac, x1r, y1r, x2r, y2r,
                     ar).astype(jnp.float32)


def _nms_kernel(x1_ref, y1_ref, x2_ref, y2_ref, s_ref, out_ref):
    out_ref[...] = jnp.zeros((_OUT_ROWS, 8), jnp.float32)

    ii = lax.broadcasted_iota(jnp.int32, (_T, _T), 0)
    jj = lax.broadcasted_iota(jnp.int32, (_T, _T), 1)
    strict_ut = (ii < jj).astype(jnp.float32)
    incl_ut = (ii <= jj).astype(jnp.float32)
    ident = (ii == jj).astype(jnp.float32)
    row_iota = lax.broadcasted_iota(jnp.int32, (_WIN, 1), 0).astype(
        jnp.float32)
    lane_iota = lax.broadcasted_iota(jnp.int32, (1, _T), 1)

    def tile_body(t, base):
        off = t * _T
        xt1 = jnp.clip(x1_ref[:, pl.ds(off, _T)], 0.0, _IM - 1.0)
        yt1 = jnp.clip(y1_ref[:, pl.ds(off, _T)], 0.0, _IM - 1.0)
        xt2 = jnp.clip(x2_ref[:, pl.ds(off, _T)], 0.0, _IM - 1.0)
        yt2 = jnp.clip(y2_ref[:, pl.ds(off, _T)], 0.0, _IM - 1.0)
        at = (xt2 - xt1 + 1.0) * (yt2 - yt1 + 1.0)
        st = s_ref[:, pl.ds(off, _T)]

        stacked8 = jnp.concatenate(
            [st, xt1, yt1, xt2, yt2, at, jnp.zeros((2, _T), jnp.float32)],
            axis=0)
        cols8 = lax.dot_general(ident, stacked8, (((1,), (1,)), ((), ())),
                                preferred_element_type=jnp.float32,
                                precision=_HI)
        x1c = cols8[:, 1:2]
        y1c = cols8[:, 2:3]
        x2c = cols8[:, 3:4]
        y2c = cols8[:, 4:5]
        ac = cols8[:, 5:6]

        def pre_body(c, acc):
            poff = c * _PW
            px1 = out_ref[pl.ds(poff, _PW), 1:2]
            py1 = out_ref[pl.ds(poff, _PW), 2:3]
            px2 = out_ref[pl.ds(poff, _PW), 3:4]
            py2 = out_ref[pl.ds(poff, _PW), 4:5]
            pa = out_ref[pl.ds(poff, _PW), 5:6]
            s_c = _iou_mask(px1, py1, px2, py2, pa, xt1, yt1, xt2, yt2, at)
            return jnp.where(jnp.any(s_c, axis=0, keepdims=True), 1.0, acc)

        sup = lax.fori_loop(0, (base + _PW - 1) // _PW, pre_body,
                            jnp.zeros((1, _T), jnp.float32))
        a_mask = jnp.where(
            (sup > 0.0) | ((lane_iota + off) >= _N), 0.0, 1.0)

        s_mat = _iou_gt(x1c, y1c, x2c, y2c, ac, xt1, yt1, xt2, yt2, at)
        s_mat = s_mat * strict_ut

        def fp_cond(c):
            return c[1]

        def fp_once(k):
            sup_k = lax.dot_general(k, s_mat, _DN,
                                    preferred_element_type=jnp.float32,
                                    precision=_HI)
            return jnp.where(sup_k > 0.0, 0.0, a_mask)

        def fp_body(c):
            k, _ = c
            nk = fp_once(fp_once(k))
            return nk, jnp.any(nk != k)

        keep, _ = lax.while_loop(fp_cond, fp_body, (a_mask, jnp.bool_(True)))

        pos_incl = lax.dot_general(keep, incl_ut, _DN,
                                   preferred_element_type=jnp.float32,
                                   precision=_HI)
        cnt = jnp.sum(keep).astype(jnp.int32)
        posf = base.astype(jnp.float32) + pos_incl - 1.0

        base_al = (jnp.minimum(base, _TOPN) // 8) * 8
        rel = posf - base_al.astype(jnp.float32)
        oh = ((row_iota == rel) & (keep > 0.0)
              & (posf < float(_TOPN))).astype(jnp.float32)
        upd = lax.dot_general(oh, cols8, _DN,
                              preferred_element_type=jnp.float32,
                              precision=_HI)
        cur = out_ref[pl.ds(base_al, _WIN), :]
        out_ref[pl.ds(base_al, _WIN), :] = cur + upd
        return base + cnt

    def tile_step(t, base):
        return lax.cond(base < _TOPN, lambda: tile_body(t, base),
                        lambda: base)

    lax.fori_loop(0, _NT, tile_step, jnp.int32(0))


def kernel(boxes, scores):
    _, x1, y1, x2, y2, s = jax.lax.sort(
        (-scores, boxes[:, 0], boxes[:, 1], boxes[:, 2], boxes[:, 3],
         scores),
        num_keys=1, is_stable=True)
    pad = _NPAD - _N
    x1 = jnp.pad(x1, ((0, pad),))[None, :]
    y1 = jnp.pad(y1, ((0, pad),))[None, :]
    x2 = jnp.pad(x2, ((0, pad),))[None, :]
    y2 = jnp.pad(y2, ((0, pad),))[None, :]
    s = jnp.pad(s, ((0, pad),))[None, :]
    out = pl.pallas_call(
        _nms_kernel,
        out_shape=jax.ShapeDtypeStruct((_OUT_ROWS, 8), jnp.float32),
    )(x1, y1, x2, y2, s)
    return out[:_TOPN, :5]


# speedup vs baseline: 1.0976x; 1.0900x over previous
import jax
import jax.numpy as jnp
from jax import lax
from jax.experimental import pallas as pl

_N = 5000
_NPAD = 5120
_T = 128
_NT = _NPAD // _T
_TOPN = 2000
_TH = 0.7
_IM = 512.0
_WIN = 136
_PW = 256
_OUT_ROWS = 2304

_DN = (((1,), (0,)), ((), ()))


def _iou_mask(x1c, y1c, x2c, y2c, ac, x1r, y1r, x2r, y2r, ar):
    xx1 = jnp.maximum(x1c, x1r)
    yy1 = jnp.maximum(y1c, y1r)
    xx2 = jnp.minimum(x2c, x2r)
    yy2 = jnp.minimum(y2c, y2r)
    w = jnp.clip(xx2 - xx1 + 1.0, 0.0)
    h = jnp.clip(yy2 - yy1 + 1.0, 0.0)
    inter = w * h
    iou = inter / (ac + ar - inter)
    return iou > _TH


def _nms_kernel(x1_ref, y1_ref, x2_ref, y2_ref, s_ref, out_ref):
    out_ref[...] = jnp.zeros((_OUT_ROWS, 8), jnp.float32)

    ii = lax.broadcasted_iota(jnp.int32, (_T, _T), 0)
    jj = lax.broadcasted_iota(jnp.int32, (_T, _T), 1)
    incl_ut = (ii <= jj).astype(jnp.bfloat16)
    ident = (ii == jj).astype(jnp.float32)
    row_iota = lax.broadcasted_iota(jnp.int32, (_WIN, 1), 0).astype(
        jnp.float32)
    lane_iota = lax.broadcasted_iota(jnp.int32, (1, _T), 1)

    def tile_body(t, base):
        off = t * _T
        xt1 = jnp.clip(x1_ref[:, pl.ds(off, _T)], 0.0, _IM - 1.0)
        yt1 = jnp.clip(y1_ref[:, pl.ds(off, _T)], 0.0, _IM - 1.0)
        xt2 = jnp.clip(x2_ref[:, pl.ds(off, _T)], 0.0, _IM - 1.0)
        yt2 = jnp.clip(y2_ref[:, pl.ds(off, _T)], 0.0, _IM - 1.0)
        at = (xt2 - xt1 + 1.0) * (yt2 - yt1 + 1.0)
        st = s_ref[:, pl.ds(off, _T)]

        stacked8 = jnp.concatenate(
            [st, xt1, yt1, xt2, yt2, at, jnp.zeros((2, _T), jnp.float32)],
            axis=0)
        cols8 = lax.dot_general(ident, stacked8, (((1,), (1,)), ((), ())),
                                preferred_element_type=jnp.float32,
                                precision=lax.Precision.HIGHEST)
        x1c = cols8[:, 1:2]
        y1c = cols8[:, 2:3]
        x2c = cols8[:, 3:4]
        y2c = cols8[:, 4:5]
        ac = cols8[:, 5:6]

        def pre_body(c, acc):
            poff = c * _PW
            px1 = out_ref[pl.ds(poff, _PW), 1:2]
            py1 = out_ref[pl.ds(poff, _PW), 2:3]
            px2 = out_ref[pl.ds(poff, _PW), 3:4]
            py2 = out_ref[pl.ds(poff, _PW), 4:5]
            pa = out_ref[pl.ds(poff, _PW), 5:6]
            s_c = _iou_mask(px1, py1, px2, py2, pa, xt1, yt1, xt2, yt2, at)
            return jnp.where(jnp.any(s_c, axis=0, keepdims=True), 1.0, acc)

        sup = lax.fori_loop(0, (base + _PW - 1) // _PW, pre_body,
                            jnp.zeros((1, _T), jnp.float32))
        a_mask = jnp.where(
            (sup > 0.0) | ((lane_iota + off) >= _N), 0.0, 1.0)

        s_mat = (_iou_mask(x1c, y1c, x2c, y2c, ac, xt1, yt1, xt2, yt2, at)
                 & (ii < jj)).astype(jnp.float32).astype(jnp.bfloat16)

        def fp_cond(c):
            return c[1]

        def fp_once(k):
            sup_k = lax.dot_general(k.astype(jnp.bfloat16), s_mat, _DN,
                                    preferred_element_type=jnp.float32)
            return jnp.where(sup_k > 0.0, 0.0, a_mask)

        def fp_body(c):
            k, _ = c
            nk = fp_once(fp_once(k))
            return nk, jnp.any(nk != k)

        keep, _ = lax.while_loop(fp_cond, fp_body, (a_mask, jnp.bool_(True)))

        pos_incl = lax.dot_general(keep.astype(jnp.bfloat16), incl_ut, _DN,
                                   preferred_element_type=jnp.float32)
        cnt = jnp.sum(pos_incl[:, _T - 1:_T]).astype(jnp.int32)
        posf = base.astype(jnp.float32) + pos_incl - 1.0

        base_al = (jnp.minimum(base, _TOPN) // 8) * 8
        rel = posf - base_al.astype(jnp.float32)
        keepf = keep.astype(jnp.float32)
        oh = ((row_iota == rel) & (keepf > 0.0)
              & (posf < float(_TOPN))).astype(jnp.float32)
        upd = lax.dot_general(oh, cols8, _DN,
                              preferred_element_type=jnp.float32,
                              precision=lax.Precision.HIGHEST)
        cur = out_ref[pl.ds(base_al, _WIN), :]
        out_ref[pl.ds(base_al, _WIN), :] = cur + upd
        return base + cnt

    def tile_step(t, base):
        return lax.cond(base < _TOPN, lambda: tile_body(t, base),
                        lambda: base)

    lax.fori_loop(0, _NT, tile_step, jnp.int32(0))


def kernel(boxes, scores):
    _, x1, y1, x2, y2, s = jax.lax.sort(
        (-scores, boxes[:, 0], boxes[:, 1], boxes[:, 2], boxes[:, 3],
         scores),
        num_keys=1, is_stable=True)
    pad = _NPAD - _N
    x1 = jnp.pad(x1, ((0, pad),))[None, :]
    y1 = jnp.pad(y1, ((0, pad),))[None, :]
    x2 = jnp.pad(x2, ((0, pad),))[None, :]
    y2 = jnp.pad(y2, ((0, pad),))[None, :]
    s = jnp.pad(s, ((0, pad),))[None, :]
    out = pl.pallas_call(
        _nms_kernel,
        out_shape=jax.ShapeDtypeStruct((_OUT_ROWS, 8), jnp.float32),
    )(x1, y1, x2, y2, s)
    return out[:_TOPN, :5]


# speedup vs baseline: 1.4973x; 1.3641x over previous
import jax
import jax.numpy as jnp
from jax import lax
from jax.experimental import pallas as pl

_N = 5000
_NPAD = 5120
_T = 256
_NT = _NPAD // _T
_TOPN = 2000
_TH = 0.7
_IM = 512.0
_WIN = 264
_PW = 256
_OUT_ROWS = 2304

_DN = (((1,), (0,)), ((), ()))


def _iou_mask(x1c, y1c, x2c, y2c, ac, x1r, y1r, x2r, y2r, ar):
    xx1 = jnp.maximum(x1c, x1r)
    yy1 = jnp.maximum(y1c, y1r)
    xx2 = jnp.minimum(x2c, x2r)
    yy2 = jnp.minimum(y2c, y2r)
    w = jnp.clip(xx2 - xx1 + 1.0, 0.0)
    h = jnp.clip(yy2 - yy1 + 1.0, 0.0)
    inter = w * h
    iou = inter / (ac + ar - inter)
    return iou > _TH


def _nms_kernel(x1_ref, y1_ref, x2_ref, y2_ref, s_ref, out_ref):
    out_ref[...] = jnp.zeros((_OUT_ROWS, 8), jnp.float32)

    ii = lax.broadcasted_iota(jnp.int32, (_T, _T), 0)
    jj = lax.broadcasted_iota(jnp.int32, (_T, _T), 1)
    incl_ut = (ii <= jj).astype(jnp.bfloat16)
    ident = (ii == jj).astype(jnp.float32)
    row_iota = lax.broadcasted_iota(jnp.int32, (_WIN, 1), 0).astype(
        jnp.float32)
    lane_iota = lax.broadcasted_iota(jnp.int32, (1, _T), 1)

    def tile_body(t, base):
        off = t * _T
        xt1 = jnp.clip(x1_ref[:, pl.ds(off, _T)], 0.0, _IM - 1.0)
        yt1 = jnp.clip(y1_ref[:, pl.ds(off, _T)], 0.0, _IM - 1.0)
        xt2 = jnp.clip(x2_ref[:, pl.ds(off, _T)], 0.0, _IM - 1.0)
        yt2 = jnp.clip(y2_ref[:, pl.ds(off, _T)], 0.0, _IM - 1.0)
        at = (xt2 - xt1 + 1.0) * (yt2 - yt1 + 1.0)
        st = s_ref[:, pl.ds(off, _T)]

        stacked8 = jnp.concatenate(
            [st, xt1, yt1, xt2, yt2, at, jnp.zeros((2, _T), jnp.float32)],
            axis=0)
        cols8 = lax.dot_general(ident, stacked8, (((1,), (1,)), ((), ())),
                                preferred_element_type=jnp.float32,
                                precision=lax.Precision.HIGHEST)
        x1c = cols8[:, 1:2]
        y1c = cols8[:, 2:3]
        x2c = cols8[:, 3:4]
        y2c = cols8[:, 4:5]
        ac = cols8[:, 5:6]

        def pre_body(c, acc):
            poff = c * _PW
            px1 = out_ref[pl.ds(poff, _PW), 1:2]
            py1 = out_ref[pl.ds(poff, _PW), 2:3]
            px2 = out_ref[pl.ds(poff, _PW), 3:4]
            py2 = out_ref[pl.ds(poff, _PW), 4:5]
            pa = out_ref[pl.ds(poff, _PW), 5:6]
            s_c = _iou_mask(px1, py1, px2, py2, pa, xt1, yt1, xt2, yt2, at)
            return jnp.where(jnp.any(s_c, axis=0, keepdims=True), 1.0, acc)

        sup = lax.fori_loop(0, (base + _PW - 1) // _PW, pre_body,
                            jnp.zeros((1, _T), jnp.float32))
        a_mask = jnp.where(
            (sup > 0.0) | ((lane_iota + off) >= _N), 0.0, 1.0)

        s_mat = (_iou_mask(x1c, y1c, x2c, y2c, ac, xt1, yt1, xt2, yt2, at)
                 & (ii < jj)).astype(jnp.float32).astype(jnp.bfloat16)

        def fp_cond(c):
            return c[1]

        def fp_once(k):
            sup_k = lax.dot_general(k.astype(jnp.bfloat16), s_mat, _DN,
                                    preferred_element_type=jnp.float32)
            return jnp.where(sup_k > 0.0, 0.0, a_mask)

        def fp_body(c):
            k, _ = c
            nk = fp_once(fp_once(k))
            return nk, jnp.any(nk != k)

        keep, _ = lax.while_loop(fp_cond, fp_body, (a_mask, jnp.bool_(True)))

        pos_incl = lax.dot_general(keep.astype(jnp.bfloat16), incl_ut, _DN,
                                   preferred_element_type=jnp.float32)
        cnt = jnp.sum(pos_incl[:, _T - 1:_T]).astype(jnp.int32)
        posf = base.astype(jnp.float32) + pos_incl - 1.0

        base_al = (jnp.minimum(base, _TOPN) // 8) * 8
        rel = posf - base_al.astype(jnp.float32)
        keepf = keep.astype(jnp.float32)
        oh = ((row_iota == rel) & (keepf > 0.0)
              & (posf < float(_TOPN))).astype(jnp.float32)
        upd = lax.dot_general(oh, cols8, _DN,
                              preferred_element_type=jnp.float32,
                              precision=lax.Precision.HIGHEST)
        cur = out_ref[pl.ds(base_al, _WIN), :]
        out_ref[pl.ds(base_al, _WIN), :] = cur + upd
        return base + cnt

    def tile_step(t, base):
        return lax.cond(base < _TOPN, lambda: tile_body(t, base),
                        lambda: base)

    lax.fori_loop(0, _NT, tile_step, jnp.int32(0))


def kernel(boxes, scores):
    _, x1, y1, x2, y2, s = jax.lax.sort(
        (-scores, boxes[:, 0], boxes[:, 1], boxes[:, 2], boxes[:, 3],
         scores),
        num_keys=1, is_stable=True)
    pad = _NPAD - _N
    x1 = jnp.pad(x1, ((0, pad),))[None, :]
    y1 = jnp.pad(y1, ((0, pad),))[None, :]
    x2 = jnp.pad(x2, ((0, pad),))[None, :]
    y2 = jnp.pad(y2, ((0, pad),))[None, :]
    s = jnp.pad(s, ((0, pad),))[None, :]
    out = pl.pallas_call(
        _nms_kernel,
        out_shape=jax.ShapeDtypeStruct((_OUT_ROWS, 8), jnp.float32),
    )(x1, y1, x2, y2, s)
    return out[:_TOPN, :5]


# speedup vs baseline: 1.5337x; 1.0243x over previous
import jax
import jax.numpy as jnp
from jax import lax
from jax.experimental import pallas as pl

_N = 5000
_NPAD = 5120
_T = 512
_NT = _NPAD // _T
_TOPN = 2000
_TH = 0.7
_IM = 512.0
_WIN = 520
_PW = 256
_OUT_ROWS = 2560

_DN = (((1,), (0,)), ((), ()))


def _iou_mask(x1c, y1c, x2c, y2c, ac, x1r, y1r, x2r, y2r, ar):
    xx1 = jnp.maximum(x1c, x1r)
    yy1 = jnp.maximum(y1c, y1r)
    xx2 = jnp.minimum(x2c, x2r)
    yy2 = jnp.minimum(y2c, y2r)
    w = jnp.clip(xx2 - xx1 + 1.0, 0.0)
    h = jnp.clip(yy2 - yy1 + 1.0, 0.0)
    inter = w * h
    iou = inter / (ac + ar - inter)
    return iou > _TH


def _nms_kernel(x1_ref, y1_ref, x2_ref, y2_ref, s_ref, out_ref):
    out_ref[...] = jnp.zeros((_OUT_ROWS, 8), jnp.float32)

    ii = lax.broadcasted_iota(jnp.int32, (_T, _T), 0)
    jj = lax.broadcasted_iota(jnp.int32, (_T, _T), 1)
    incl_ut = (ii <= jj).astype(jnp.bfloat16)
    ident = (ii == jj).astype(jnp.float32)
    row_iota = lax.broadcasted_iota(jnp.int32, (_WIN, 1), 0).astype(
        jnp.float32)
    lane_iota = lax.broadcasted_iota(jnp.int32, (1, _T), 1)

    def tile_body(t, base):
        off = t * _T
        xt1 = jnp.clip(x1_ref[:, pl.ds(off, _T)], 0.0, _IM - 1.0)
        yt1 = jnp.clip(y1_ref[:, pl.ds(off, _T)], 0.0, _IM - 1.0)
        xt2 = jnp.clip(x2_ref[:, pl.ds(off, _T)], 0.0, _IM - 1.0)
        yt2 = jnp.clip(y2_ref[:, pl.ds(off, _T)], 0.0, _IM - 1.0)
        at = (xt2 - xt1 + 1.0) * (yt2 - yt1 + 1.0)
        st = s_ref[:, pl.ds(off, _T)]

        stacked8 = jnp.concatenate(
            [st, xt1, yt1, xt2, yt2, at, jnp.zeros((2, _T), jnp.float32)],
            axis=0)
        cols8 = lax.dot_general(ident, stacked8, (((1,), (1,)), ((), ())),
                                preferred_element_type=jnp.float32,
                                precision=lax.Precision.HIGHEST)
        x1c = cols8[:, 1:2]
        y1c = cols8[:, 2:3]
        x2c = cols8[:, 3:4]
        y2c = cols8[:, 4:5]
        ac = cols8[:, 5:6]

        def pre_body(c, acc):
            poff = c * _PW
            px1 = out_ref[pl.ds(poff, _PW), 1:2]
            py1 = out_ref[pl.ds(poff, _PW), 2:3]
            px2 = out_ref[pl.ds(poff, _PW), 3:4]
            py2 = out_ref[pl.ds(poff, _PW), 4:5]
            pa = out_ref[pl.ds(poff, _PW), 5:6]
            s_c = _iou_mask(px1, py1, px2, py2, pa, xt1, yt1, xt2, yt2, at)
            return jnp.where(jnp.any(s_c, axis=0, keepdims=True), 1.0, acc)

        sup = lax.fori_loop(0, (base + _PW - 1) // _PW, pre_body,
                            jnp.zeros((1, _T), jnp.float32))
        a_mask = jnp.where(
            (sup > 0.0) | ((lane_iota + off) >= _N), 0.0, 1.0)

        s_mat = (_iou_mask(x1c, y1c, x2c, y2c, ac, xt1, yt1, xt2, yt2, at)
                 & (ii < jj)).astype(jnp.float32).astype(jnp.bfloat16)

        def fp_cond(c):
            return c[1]

        def fp_once(k):
            sup_k = lax.dot_general(k.astype(jnp.bfloat16), s_mat, _DN,
                                    preferred_element_type=jnp.float32)
            return jnp.where(sup_k > 0.0, 0.0, a_mask)

        def fp_body(c):
            k, _ = c
            nk = fp_once(fp_once(k))
            return nk, jnp.any(nk != k)

        keep, _ = lax.while_loop(fp_cond, fp_body, (a_mask, jnp.bool_(True)))

        pos_incl = lax.dot_general(keep.astype(jnp.bfloat16), incl_ut, _DN,
                                   preferred_element_type=jnp.float32)
        cnt = jnp.sum(pos_incl[:, _T - 1:_T]).astype(jnp.int32)
        posf = base.astype(jnp.float32) + pos_incl - 1.0

        base_al = (jnp.minimum(base, _TOPN) // 8) * 8
        rel = posf - base_al.astype(jnp.float32)
        keepf = keep.astype(jnp.float32)
        oh = ((row_iota == rel) & (keepf > 0.0)
              & (posf < float(_TOPN))).astype(jnp.float32)
        upd = lax.dot_general(oh, cols8, _DN,
                              preferred_element_type=jnp.float32,
                              precision=lax.Precision.HIGHEST)
        cur = out_ref[pl.ds(base_al, _WIN), :]
        out_ref[pl.ds(base_al, _WIN), :] = cur + upd
        return base + cnt

    def tile_step(t, base):
        return lax.cond(base < _TOPN, lambda: tile_body(t, base),
                        lambda: base)

    lax.fori_loop(0, _NT, tile_step, jnp.int32(0))


def kernel(boxes, scores):
    _, x1, y1, x2, y2, s = jax.lax.sort(
        (-scores, boxes[:, 0], boxes[:, 1], boxes[:, 2], boxes[:, 3],
         scores),
        num_keys=1, is_stable=True)
    pad = _NPAD - _N
    x1 = jnp.pad(x1, ((0, pad),))[None, :]
    y1 = jnp.pad(y1, ((0, pad),))[None, :]
    x2 = jnp.pad(x2, ((0, pad),))[None, :]
    y2 = jnp.pad(y2, ((0, pad),))[None, :]
    s = jnp.pad(s, ((0, pad),))[None, :]
    out = pl.pallas_call(
        _nms_kernel,
        out_shape=jax.ShapeDtypeStruct((_OUT_ROWS, 8), jnp.float32),
    )(x1, y1, x2, y2, s)
    return out[:_TOPN, :5]


# speedup vs baseline: 1.6892x; 1.1014x over previous
import jax
import jax.numpy as jnp
from jax import lax
from jax.experimental import pallas as pl

_N = 5000
_NPAD = 5120
_T = 512
_NT = _NPAD // _T
_TOPN = 2000
_TH = 0.7
_IM = 512.0
_WIN = 520
_PW = 256
_OUT_ROWS = 2560

_DN = (((1,), (0,)), ((), ()))


def _iou_mask(x1c, y1c, x2c, y2c, ac, x1r, y1r, x2r, y2r, ar):
    xx1 = jnp.maximum(x1c, x1r)
    yy1 = jnp.maximum(y1c, y1r)
    xx2 = jnp.minimum(x2c, x2r)
    yy2 = jnp.minimum(y2c, y2r)
    w = jnp.clip(xx2 - xx1 + 1.0, 0.0)
    h = jnp.clip(yy2 - yy1 + 1.0, 0.0)
    inter = w * h
    iou = inter / (ac + ar - inter)
    return iou > _TH


def _nms_kernel(x1_ref, y1_ref, x2_ref, y2_ref, s_ref, out_ref):
    out_ref[...] = jnp.zeros((_OUT_ROWS, 8), jnp.float32)

    ii = lax.broadcasted_iota(jnp.int32, (_T, _T), 0)
    jj = lax.broadcasted_iota(jnp.int32, (_T, _T), 1)
    incl_ut = (ii <= jj).astype(jnp.bfloat16)
    ident = (ii == jj).astype(jnp.float32)
    row_iota = lax.broadcasted_iota(jnp.int32, (_WIN, 1), 0).astype(
        jnp.float32)
    lane_iota = lax.broadcasted_iota(jnp.int32, (1, _T), 1)

    def tile_body(t, base):
        off = t * _T
        xt1 = jnp.clip(x1_ref[:, pl.ds(off, _T)], 0.0, _IM - 1.0)
        yt1 = jnp.clip(y1_ref[:, pl.ds(off, _T)], 0.0, _IM - 1.0)
        xt2 = jnp.clip(x2_ref[:, pl.ds(off, _T)], 0.0, _IM - 1.0)
        yt2 = jnp.clip(y2_ref[:, pl.ds(off, _T)], 0.0, _IM - 1.0)
        at = (xt2 - xt1 + 1.0) * (yt2 - yt1 + 1.0)
        st = s_ref[:, pl.ds(off, _T)]

        stacked8 = jnp.concatenate(
            [st, xt1, yt1, xt2, yt2, at, jnp.zeros((2, _T), jnp.float32)],
            axis=0)
        cols8 = jnp.transpose(stacked8, (1, 0))
        x1c = cols8[:, 1:2]
        y1c = cols8[:, 2:3]
        x2c = cols8[:, 3:4]
        y2c = cols8[:, 4:5]
        ac = cols8[:, 5:6]

        def pre_body(c, acc):
            poff = c * _PW
            px1 = out_ref[pl.ds(poff, _PW), 1:2]
            py1 = out_ref[pl.ds(poff, _PW), 2:3]
            px2 = out_ref[pl.ds(poff, _PW), 3:4]
            py2 = out_ref[pl.ds(poff, _PW), 4:5]
            pa = out_ref[pl.ds(poff, _PW), 5:6]
            s_c = _iou_mask(px1, py1, px2, py2, pa, xt1, yt1, xt2, yt2, at)
            return jnp.where(jnp.any(s_c, axis=0, keepdims=True), 1.0, acc)

        sup = lax.fori_loop(0, (base + _PW - 1) // _PW, pre_body,
                            jnp.zeros((1, _T), jnp.float32))
        a_mask = jnp.where(
            (sup > 0.0) | ((lane_iota + off) >= _N), 0.0, 1.0)

        s_mat = (_iou_mask(x1c, y1c, x2c, y2c, ac, xt1, yt1, xt2, yt2, at)
                 & (ii < jj)).astype(jnp.float32).astype(jnp.bfloat16)

        def fp_cond(c):
            return c[1]

        def fp_once(k):
            sup_k = lax.dot_general(k.astype(jnp.bfloat16), s_mat, _DN,
                                    preferred_element_type=jnp.float32)
            return jnp.where(sup_k > 0.0, 0.0, a_mask)

        def fp_body(c):
            k, _ = c
            nk = fp_once(fp_once(k))
            return nk, jnp.any(nk != k)

        keep, _ = lax.while_loop(fp_cond, fp_body, (a_mask, jnp.bool_(True)))

        pos_incl = lax.dot_general(keep.astype(jnp.bfloat16), incl_ut, _DN,
                                   preferred_element_type=jnp.float32)
        cnt = jnp.sum(pos_incl[:, _T - 1:_T]).astype(jnp.int32)
        posf = base.astype(jnp.float32) + pos_incl - 1.0

        base_al = (jnp.minimum(base, _TOPN) // 8) * 8
        rel = posf - base_al.astype(jnp.float32)
        keepf = keep.astype(jnp.float32)
        oh = ((row_iota == rel) & (keepf > 0.0)
              & (posf < float(_TOPN))).astype(jnp.float32)
        upd = lax.dot_general(oh, cols8, _DN,
                              preferred_element_type=jnp.float32,
                              precision=lax.Precision.HIGHEST)
        cur = out_ref[pl.ds(base_al, _WIN), :]
        out_ref[pl.ds(base_al, _WIN), :] = cur + upd
        return base + cnt

    def tile_step(t, base):
        return lax.cond(base < _TOPN, lambda: tile_body(t, base),
                        lambda: base)

    lax.fori_loop(0, _NT, tile_step, jnp.int32(0))


def kernel(boxes, scores):
    _, x1, y1, x2, y2, s = jax.lax.sort(
        (-scores, boxes[:, 0], boxes[:, 1], boxes[:, 2], boxes[:, 3],
         scores),
        num_keys=1, is_stable=True)
    pad = _NPAD - _N
    x1 = jnp.pad(x1, ((0, pad),))[None, :]
    y1 = jnp.pad(y1, ((0, pad),))[None, :]
    x2 = jnp.pad(x2, ((0, pad),))[None, :]
    y2 = jnp.pad(y2, ((0, pad),))[None, :]
    s = jnp.pad(s, ((0, pad),))[None, :]
    out = pl.pallas_call(
        _nms_kernel,
        out_shape=jax.ShapeDtypeStruct((_OUT_ROWS, 8), jnp.float32),
    )(x1, y1, x2, y2, s)
    return out[:_TOPN, :5]


# speedup vs baseline: 1.7565x; 1.0399x over previous
import jax
import jax.numpy as jnp
from jax import lax
from jax.experimental import pallas as pl

_N = 5000
_NPAD = 5120
_T = 512
_NT = _NPAD // _T
_TOPN = 2000
_TH = 0.7
_IM = 512.0
_WIN = 520
_PW = 256
_OUT_ROWS = 2560

_DN = (((1,), (0,)), ((), ()))


def _iou_mask(x1c, y1c, x2c, y2c, ac, x1r, y1r, x2r, y2r, ar):
    xx1 = jnp.maximum(x1c, x1r)
    yy1 = jnp.maximum(y1c, y1r)
    xx2 = jnp.minimum(x2c, x2r)
    yy2 = jnp.minimum(y2c, y2r)
    w = jnp.clip(xx2 - xx1 + 1.0, 0.0)
    h = jnp.clip(yy2 - yy1 + 1.0, 0.0)
    inter = w * h
    iou = inter / (ac + ar - inter)
    return iou > _TH


def _nms_kernel(x1_ref, y1_ref, x2_ref, y2_ref, s_ref, out_ref):
    out_ref[...] = jnp.zeros((_OUT_ROWS, 8), jnp.float32)

    ii = lax.broadcasted_iota(jnp.int32, (_T, _T), 0)
    jj = lax.broadcasted_iota(jnp.int32, (_T, _T), 1)
    incl_ut = (ii <= jj).astype(jnp.bfloat16)
    ident = (ii == jj).astype(jnp.float32)
    row_iota = lax.broadcasted_iota(jnp.int32, (_WIN, 1), 0).astype(
        jnp.float32)
    lane_iota = lax.broadcasted_iota(jnp.int32, (1, _T), 1)

    def tile_body(t, base):
        off = t * _T
        xt1 = jnp.clip(x1_ref[:, pl.ds(off, _T)], 0.0, _IM - 1.0)
        yt1 = jnp.clip(y1_ref[:, pl.ds(off, _T)], 0.0, _IM - 1.0)
        xt2 = jnp.clip(x2_ref[:, pl.ds(off, _T)], 0.0, _IM - 1.0)
        yt2 = jnp.clip(y2_ref[:, pl.ds(off, _T)], 0.0, _IM - 1.0)
        at = (xt2 - xt1 + 1.0) * (yt2 - yt1 + 1.0)
        st = s_ref[:, pl.ds(off, _T)]

        stacked8 = jnp.concatenate(
            [st, xt1, yt1, xt2, yt2, at, jnp.zeros((2, _T), jnp.float32)],
            axis=0)
        cols8 = jnp.transpose(stacked8, (1, 0))
        x1c = cols8[:, 1:2]
        y1c = cols8[:, 2:3]
        x2c = cols8[:, 3:4]
        y2c = cols8[:, 4:5]
        ac = cols8[:, 5:6]

        def pre_body(c, acc):
            poff = c * _PW
            px1 = out_ref[pl.ds(poff, _PW), 1:2]
            py1 = out_ref[pl.ds(poff, _PW), 2:3]
            px2 = out_ref[pl.ds(poff, _PW), 3:4]
            py2 = out_ref[pl.ds(poff, _PW), 4:5]
            pa = out_ref[pl.ds(poff, _PW), 5:6]
            s_c = _iou_mask(px1, py1, px2, py2, pa, xt1, yt1, xt2, yt2, at)
            return jnp.where(jnp.any(s_c, axis=0, keepdims=True), 1.0, acc)

        sup = lax.fori_loop(0, (base + _PW - 1) // _PW, pre_body,
                            jnp.zeros((1, _T), jnp.float32))
        a_mask = jnp.where(
            (sup > 0.0) | ((lane_iota + off) >= _N), 0.0, 1.0)

        s_mat = (_iou_mask(x1c, y1c, x2c, y2c, ac, xt1, yt1, xt2, yt2, at)
                 & (ii < jj)).astype(jnp.float32).astype(jnp.bfloat16)

        def fp_cond(c):
            return c[1]

        def fp_once(k):
            sup_k = lax.dot_general(k.astype(jnp.bfloat16), s_mat, _DN,
                                    preferred_element_type=jnp.float32)
            return jnp.where(sup_k > 0.0, 0.0, a_mask)

        def fp_body(c):
            k, _ = c
            nk = fp_once(fp_once(k))
            return nk, jnp.any(nk != k)

        keep, _ = lax.while_loop(fp_cond, fp_body, (a_mask, jnp.bool_(True)))

        pos_incl = lax.dot_general(keep.astype(jnp.bfloat16), incl_ut, _DN,
                                   preferred_element_type=jnp.float32)
        cnt = jnp.sum(pos_incl[:, _T - 1:_T]).astype(jnp.int32)
        posf = base.astype(jnp.float32) + pos_incl - 1.0

        base_al = (jnp.minimum(base, _TOPN) // 8) * 8
        rel = posf - base_al.astype(jnp.float32)
        keepf = keep.astype(jnp.float32)
        oh = ((row_iota == rel) & (keepf > 0.0)
              & (posf < float(_TOPN))).astype(jnp.float32)
        oh_b = oh.astype(jnp.bfloat16)
        c1 = cols8.astype(jnp.bfloat16)
        r1 = cols8 - c1.astype(jnp.float32)
        c2 = r1.astype(jnp.bfloat16)
        c3 = (r1 - c2.astype(jnp.float32)).astype(jnp.bfloat16)
        upd = (lax.dot_general(oh_b, c1, _DN,
                               preferred_element_type=jnp.float32)
               + lax.dot_general(oh_b, c2, _DN,
                                 preferred_element_type=jnp.float32)
               + lax.dot_general(oh_b, c3, _DN,
                                 preferred_element_type=jnp.float32))
        cur = out_ref[pl.ds(base_al, _WIN), :]
        out_ref[pl.ds(base_al, _WIN), :] = cur + upd
        return base + cnt

    def tile_step(t, base):
        return lax.cond(base < _TOPN, lambda: tile_body(t, base),
                        lambda: base)

    lax.fori_loop(0, _NT, tile_step, jnp.int32(0))


def kernel(boxes, scores):
    _, x1, y1, x2, y2, s = jax.lax.sort(
        (-scores, boxes[:, 0], boxes[:, 1], boxes[:, 2], boxes[:, 3],
         scores),
        num_keys=1, is_stable=True)
    pad = _NPAD - _N
    x1 = jnp.pad(x1, ((0, pad),))[None, :]
    y1 = jnp.pad(y1, ((0, pad),))[None, :]
    x2 = jnp.pad(x2, ((0, pad),))[None, :]
    y2 = jnp.pad(y2, ((0, pad),))[None, :]
    s = jnp.pad(s, ((0, pad),))[None, :]
    out = pl.pallas_call(
        _nms_kernel,
        out_shape=jax.ShapeDtypeStruct((_OUT_ROWS, 8), jnp.float32),
    )(x1, y1, x2, y2, s)
    return out[:_TOPN, :5]


# speedup vs baseline: 1.7951x; 1.0220x over previous
import jax
import jax.numpy as jnp
from jax import lax
from jax.experimental import pallas as pl

_N = 5000
_NPAD = 5120
_T = 512
_NT = _NPAD // _T
_TOPN = 2000
_TH = 0.7
_IM = 512.0
_WIN = 520
_PW = 512
_OUT_ROWS = 2560

_DN = (((1,), (0,)), ((), ()))


def _iou_mask(x1c, y1c, x2c, y2c, ac, x1r, y1r, x2r, y2r, ar):
    xx1 = jnp.maximum(x1c, x1r)
    yy1 = jnp.maximum(y1c, y1r)
    xx2 = jnp.minimum(x2c, x2r)
    yy2 = jnp.minimum(y2c, y2r)
    w = jnp.clip(xx2 - xx1 + 1.0, 0.0)
    h = jnp.clip(yy2 - yy1 + 1.0, 0.0)
    inter = w * h
    iou = inter / (ac + ar - inter)
    return iou > _TH


def _nms_kernel(x1_ref, y1_ref, x2_ref, y2_ref, s_ref, out_ref):
    out_ref[...] = jnp.zeros((_OUT_ROWS, 8), jnp.float32)

    ii = lax.broadcasted_iota(jnp.int32, (_T, _T), 0)
    jj = lax.broadcasted_iota(jnp.int32, (_T, _T), 1)
    incl_ut = (ii <= jj).astype(jnp.bfloat16)
    ident = (ii == jj).astype(jnp.float32)
    row_iota = lax.broadcasted_iota(jnp.int32, (_WIN, 1), 0).astype(
        jnp.float32)
    lane_iota = lax.broadcasted_iota(jnp.int32, (1, _T), 1)

    def tile_body(t, base):
        off = t * _T
        xt1 = jnp.clip(x1_ref[:, pl.ds(off, _T)], 0.0, _IM - 1.0)
        yt1 = jnp.clip(y1_ref[:, pl.ds(off, _T)], 0.0, _IM - 1.0)
        xt2 = jnp.clip(x2_ref[:, pl.ds(off, _T)], 0.0, _IM - 1.0)
        yt2 = jnp.clip(y2_ref[:, pl.ds(off, _T)], 0.0, _IM - 1.0)
        at = (xt2 - xt1 + 1.0) * (yt2 - yt1 + 1.0)
        st = s_ref[:, pl.ds(off, _T)]

        stacked8 = jnp.concatenate(
            [st, xt1, yt1, xt2, yt2, at, jnp.zeros((2, _T), jnp.float32)],
            axis=0)
        cols8 = jnp.transpose(stacked8, (1, 0))
        x1c = cols8[:, 1:2]
        y1c = cols8[:, 2:3]
        x2c = cols8[:, 3:4]
        y2c = cols8[:, 4:5]
        ac = cols8[:, 5:6]

        def pre_body(c, acc):
            poff = c * _PW
            px1 = out_ref[pl.ds(poff, _PW), 1:2]
            py1 = out_ref[pl.ds(poff, _PW), 2:3]
            px2 = out_ref[pl.ds(poff, _PW), 3:4]
            py2 = out_ref[pl.ds(poff, _PW), 4:5]
            pa = out_ref[pl.ds(poff, _PW), 5:6]
            s_c = _iou_mask(px1, py1, px2, py2, pa, xt1, yt1, xt2, yt2, at)
            return jnp.where(jnp.any(s_c, axis=0, keepdims=True), 1.0, acc)

        sup = lax.fori_loop(0, (base + _PW - 1) // _PW, pre_body,
                            jnp.zeros((1, _T), jnp.float32))
        a_mask = jnp.where(
            (sup > 0.0) | ((lane_iota + off) >= _N), 0.0, 1.0)

        s_mat = (_iou_mask(x1c, y1c, x2c, y2c, ac, xt1, yt1, xt2, yt2, at)
                 & (ii < jj)).astype(jnp.float32).astype(jnp.bfloat16)

        def fp_cond(c):
            return c[1]

        def fp_once(k):
            sup_k = lax.dot_general(k.astype(jnp.bfloat16), s_mat, _DN,
                                    preferred_element_type=jnp.float32)
            return jnp.where(sup_k > 0.0, 0.0, a_mask)

        def fp_body(c):
            k, _ = c
            nk = fp_once(fp_once(k))
            return nk, jnp.any(nk != k)

        keep, _ = lax.while_loop(fp_cond, fp_body, (a_mask, jnp.bool_(True)))

        pos_incl = lax.dot_general(keep.astype(jnp.bfloat16), incl_ut, _DN,
                                   preferred_element_type=jnp.float32)
        cnt = jnp.sum(pos_incl[:, _T - 1:_T]).astype(jnp.int32)
        posf = base.astype(jnp.float32) + pos_incl - 1.0

        base_al = (jnp.minimum(base, _TOPN) // 8) * 8
        rel = posf - base_al.astype(jnp.float32)
        keepf = keep.astype(jnp.float32)
        oh = ((row_iota == rel) & (keepf > 0.0)
              & (posf < float(_TOPN))).astype(jnp.float32)
        oh_b = oh.astype(jnp.bfloat16)
        c1 = cols8.astype(jnp.bfloat16)
        r1 = cols8 - c1.astype(jnp.float32)
        c2 = r1.astype(jnp.bfloat16)
        c3 = (r1 - c2.astype(jnp.float32)).astype(jnp.bfloat16)
        upd = (lax.dot_general(oh_b, c1, _DN,
                               preferred_element_type=jnp.float32)
               + lax.dot_general(oh_b, c2, _DN,
                                 preferred_element_type=jnp.float32)
               + lax.dot_general(oh_b, c3, _DN,
                                 preferred_element_type=jnp.float32))
        cur = out_ref[pl.ds(base_al, _WIN), :]
        out_ref[pl.ds(base_al, _WIN), :] = cur + upd
        return base + cnt

    def tile_step(t, base):
        return lax.cond(base < _TOPN, lambda: tile_body(t, base),
                        lambda: base)

    lax.fori_loop(0, _NT, tile_step, jnp.int32(0))


def kernel(boxes, scores):
    _, x1, y1, x2, y2, s = jax.lax.sort(
        (-scores, boxes[:, 0], boxes[:, 1], boxes[:, 2], boxes[:, 3],
         scores),
        num_keys=1, is_stable=True)
    pad = _NPAD - _N
    x1 = jnp.pad(x1, ((0, pad),))[None, :]
    y1 = jnp.pad(y1, ((0, pad),))[None, :]
    x2 = jnp.pad(x2, ((0, pad),))[None, :]
    y2 = jnp.pad(y2, ((0, pad),))[None, :]
    s = jnp.pad(s, ((0, pad),))[None, :]
    out = pl.pallas_call(
        _nms_kernel,
        out_shape=jax.ShapeDtypeStruct((_OUT_ROWS, 8), jnp.float32),
    )(x1, y1, x2, y2, s)
    return out[:_TOPN, :5]
